# baseline (speedup 1.0000x reference)
import jax
import jax.numpy as jnp
from jax.experimental import pallas as pl
from jax.experimental.pallas import tpu as pltpu


def _update_body(offs_ref, dst_ref, src_ref, out_ref, copy_sems, sc_sems):
    B = dst_ref.shape[0]
    Q = src_ref.shape[2]
    copies = []
    for b in range(B):
        c = pltpu.make_async_copy(dst_ref.at[b], out_ref.at[b], copy_sems.at[b])
        c.start()
        copies.append(c)
    scatters = []
    for b in range(B):
        copies[b].wait()
        off = offs_ref[b]
        c = pltpu.make_async_copy(
            src_ref.at[b], out_ref.at[b, :, pl.ds(off, Q), :], sc_sems.at[b]
        )
        c.start()
        scatters.append(c)
    for c in scatters:
        c.wait()


def kernel(dst_in, indices_in, src_in, dim_i):
    B, H, S, D = dst_in.shape
    Q = src_in.shape[2]
    offs = (indices_in + (jnp.asarray(dim_i) - 2)).astype(jnp.int32)

    grid_spec = pltpu.PrefetchScalarGridSpec(
        num_scalar_prefetch=1,
        grid=(1,),
        in_specs=[
            pl.BlockSpec(memory_space=pl.ANY),
            pl.BlockSpec(memory_space=pl.ANY),
        ],
        out_specs=pl.BlockSpec(memory_space=pl.ANY),
        scratch_shapes=[
            pltpu.SemaphoreType.DMA((B,)),
            pltpu.SemaphoreType.DMA((B,)),
        ],
    )
    return pl.pallas_call(
        _update_body,
        grid_spec=grid_spec,
        out_shape=jax.ShapeDtypeStruct(dst_in.shape, dst_in.dtype),
    )(offs, dst_in, src_in)


# speedup vs baseline: 46.0610x; 46.0610x over previous
import jax
import jax.numpy as jnp
from jax import lax
from jax.experimental import pallas as pl
from jax.experimental.pallas import tpu as pltpu
from jax.experimental.pallas import tpu_sc as plsc


def _copy_body(dst_ref, out_ref):
    out_ref[...] = dst_ref[...]


def _tc_copy(dst_in):
    B, H, S, D = dst_in.shape
    HB = 8
    return pl.pallas_call(
        _copy_body,
        grid=(B, H // HB),
        in_specs=[pl.BlockSpec((1, HB, S, D), lambda b, h: (b, h, 0, 0))],
        out_specs=pl.BlockSpec((1, HB, S, D), lambda b, h: (b, h, 0, 0)),
        out_shape=jax.ShapeDtypeStruct(dst_in.shape, dst_in.dtype),
    )(dst_in)


def _make_sc_scatter(NW, K, D):
    mesh = plsc.VectorSubcoreMesh(core_axis_name="c", subcore_axis_name="s")
    NC = mesh.num_cores

    @pl.kernel(
        mesh=mesh,
        out_type=(),
        scratch_types=[
            pltpu.VMEM((K, 128), jnp.int32),
            pltpu.VMEM((K, 128, D), jnp.float32),
            pltpu.SemaphoreType.DMA,
        ],
    )
    def sc_scatter(out_hbm, rows_hbm, src_hbm, idx_v, rows_v, sem):
        wid = lax.axis_index("s") * NC + lax.axis_index("c")
        pltpu.sync_copy(rows_hbm.at[wid], idx_v)
        pltpu.sync_copy(src_hbm.at[wid], rows_v)
        for j in range(K):
            pltpu.async_copy(rows_v.at[j], out_hbm.at[idx_v.at[j]], sem).wait()

    return sc_scatter


def kernel(dst_in, indices_in, src_in, dim_i):
    B, H, S, D = dst_in.shape
    Q = src_in.shape[2]
    offs = (indices_in + (jnp.asarray(dim_i) - 2)).astype(jnp.int32)

    info = plsc.get_sparse_core_info()
    NW = info.num_cores * info.num_subcores
    K = (B * H * Q) // (NW * 128)

    base = ((jnp.arange(B) * H)[:, None] + jnp.arange(H)[None, :]) * S
    rows = base[:, :, None] + offs[:, None, None] + jnp.arange(Q)[None, None, :]
    rows = rows.reshape(NW, K, 128).astype(jnp.int32)
    src_r = src_in.reshape(NW, K, 128, D)

    out = _tc_copy(dst_in)
    out_ref = jax.new_ref(out.reshape(B * H * S, D))
    _make_sc_scatter(NW, K, D)(out_ref, rows, src_r)
    return out_ref[...].reshape(B, H, S, D)


# speedup vs baseline: 46.0915x; 1.0007x over previous
import jax
import jax.numpy as jnp
from jax import lax
from jax.experimental import pallas as pl
from jax.experimental.pallas import tpu as pltpu
from jax.experimental.pallas import tpu_sc as plsc


def _copy_body(dst_ref, out_ref):
    out_ref[...] = dst_ref[...]


def _tc_copy(dst_in):
    B, H, S, D = dst_in.shape
    HB = 8
    return pl.pallas_call(
        _copy_body,
        grid=(B, H // HB),
        in_specs=[pl.BlockSpec((1, HB, S, D), lambda b, h: (b, h, 0, 0))],
        out_specs=pl.BlockSpec((1, HB, S, D), lambda b, h: (b, h, 0, 0)),
        out_shape=jax.ShapeDtypeStruct(dst_in.shape, dst_in.dtype),
    )(dst_in)


def _make_sc_scatter(NW, K, D):
    mesh = plsc.VectorSubcoreMesh(core_axis_name="c", subcore_axis_name="s")
    NC = mesh.num_cores

    @pl.kernel(
        mesh=mesh,
        out_type=(),
        scratch_types=[
            pltpu.VMEM((K, 128), jnp.int32),
            pltpu.VMEM((K, 128, D), jnp.float32),
            pltpu.SemaphoreType.DMA,
            pltpu.SemaphoreType.DMA,
            pltpu.SemaphoreType.DMA((K,)),
        ],
    )
    def sc_scatter(out_hbm, rows_hbm, src_hbm, idx_v, rows_v, sem_i, sem_s, sem_sc):
        wid = lax.axis_index("s") * NC + lax.axis_index("c")
        ci = pltpu.make_async_copy(rows_hbm.at[wid], idx_v, sem_i)
        cs = pltpu.make_async_copy(src_hbm.at[wid], rows_v, sem_s)
        ci.start()
        cs.start()
        ci.wait()
        cs.wait()
        scs = []
        for j in range(K):
            c = pltpu.make_async_copy(rows_v.at[j], out_hbm.at[idx_v.at[j]], sem_sc.at[j])
            c.start()
            scs.append(c)
        for c in scs:
            c.wait()

    return sc_scatter


def kernel(dst_in, indices_in, src_in, dim_i):
    B, H, S, D = dst_in.shape
    Q = src_in.shape[2]
    offs = (indices_in + (jnp.asarray(dim_i) - 2)).astype(jnp.int32)

    info = plsc.get_sparse_core_info()
    NW = info.num_cores * info.num_subcores
    K = (B * H * Q) // (NW * 128)

    base = ((jnp.arange(B) * H)[:, None] + jnp.arange(H)[None, :]) * S
    rows = base[:, :, None] + offs[:, None, None] + jnp.arange(Q)[None, None, :]
    rows = rows.reshape(NW, K, 128).astype(jnp.int32)
    src_r = src_in.reshape(NW, K, 128, D)

    out = _tc_copy(dst_in)
    out_ref = jax.new_ref(out.reshape(B * H * S, D))
    _make_sc_scatter(NW, K, D)(out_ref, rows, src_r)
    return out_ref[...].reshape(B, H, S, D)


# speedup vs baseline: 46.4910x; 1.0087x over previous
import jax
import jax.numpy as jnp
from jax import lax
from jax.experimental import pallas as pl
from jax.experimental.pallas import tpu as pltpu
from jax.experimental.pallas import tpu_sc as plsc


def _copy_body(dst_ref, out_ref):
    out_ref[...] = dst_ref[...]


def _tc_copy(dst_in):
    B, H, S, D = dst_in.shape
    HB = 8
    return pl.pallas_call(
        _copy_body,
        grid=(B, H // HB),
        in_specs=[pl.BlockSpec((1, HB, S, D), lambda b, h: (b, h, 0, 0))],
        out_specs=pl.BlockSpec((1, HB, S, D), lambda b, h: (b, h, 0, 0)),
        out_shape=jax.ShapeDtypeStruct(dst_in.shape, dst_in.dtype),
    )(dst_in)


def _make_sc_scatter(NW, K, D):
    mesh = plsc.VectorSubcoreMesh(core_axis_name="c", subcore_axis_name="s")
    NC = mesh.num_cores

    @pl.kernel(
        mesh=mesh,
        out_type=(),
        scratch_types=[
            pltpu.VMEM((K, 128), jnp.int32),
            pltpu.VMEM((K, 128, D), jnp.float32),
            pltpu.SemaphoreType.DMA,
            pltpu.SemaphoreType.DMA,
            pltpu.SemaphoreType.DMA((K,)),
        ],
    )
    def sc_scatter(out_hbm, rows_hbm, src_hbm, idx_v, rows_v, sem_i, sem_s, sem_sc):
        wid = lax.axis_index("s") * NC + lax.axis_index("c")
        ci = pltpu.make_async_copy(rows_hbm.at[wid], idx_v, sem_i)
        cs = pltpu.make_async_copy(src_hbm.at[wid], rows_v, sem_s)
        ci.start()
        cs.start()
        ci.wait()
        cs.wait()
        scs = []
        for j in range(K):
            c = pltpu.make_async_copy(rows_v.at[j], out_hbm.at[idx_v.at[j]], sem_sc.at[j])
            c.start()
            scs.append(c)
        for c in scs:
            c.wait()

    return sc_scatter


def kernel(dst_in, indices_in, src_in, dim_i):
    B, H, S, D = dst_in.shape
    Q = src_in.shape[2]
    offs = (indices_in + (jnp.asarray(dim_i) - 2)).astype(jnp.int32)

    info = plsc.get_sparse_core_info()
    NW = info.num_cores * info.num_subcores
    K = (B * H * Q) // (NW * 128)

    base = ((jnp.arange(B) * H)[:, None] + jnp.arange(H)[None, :]) * S
    rows = base[:, :, None] + offs[:, None, None] + jnp.arange(Q)[None, None, :]
    rows = rows.reshape(NW, K, 128).astype(jnp.int32)
    src_r = src_in.reshape(NW, K, 128, D)

    out_ref = jax.new_ref(dst_in.reshape(B * H * S, D))
    _make_sc_scatter(NW, K, D)(out_ref, rows, src_r)
    return out_ref[...].reshape(B, H, S, D)
